# baseline (speedup 1.0000x reference)
import functools

import jax
import jax.numpy as jnp
from jax import lax
from jax.experimental import pallas as pl
from jax.experimental.pallas import tpu as pltpu
from jax.experimental.pallas import tpu_sc as plsc

T, D, H = 8192, 768, 12
DH = D // H
KEEP = max(1, int(T * 0.1))
KP = 896
TBLK = 1024
ABLK = 1024
NW = 32
IDXF = 912
EQTRASH = T + 16

_PREC = lax.Precision.DEFAULT
_DOTF32 = jnp.float32


def _dot(a, b, dims):
    return lax.dot_general(a, b, (dims, ((), ())),
                           preferred_element_type=_DOTF32, precision=_PREC)


def _qkv_body(x_ref, w_ref, b_ref, q_ref, kv_ref, qn2_ref):
    x = x_ref[...]
    qkv = _dot(x, w_ref[...], (((1,), (0,)))) + b_ref[...]
    q = qkv[:, :D]
    for h in range(H):
        q_ref[h] = q[:, h * DH:(h + 1) * DH]
    parts = []
    for h in range(H):
        parts.append(qkv[:, D + h * DH:D + (h + 1) * DH])
        parts.append(qkv[:, 2 * D + h * DH:2 * D + (h + 1) * DH])
    kv_ref[...] = jnp.concatenate(parts, axis=1)
    col = lax.broadcasted_iota(jnp.int32, (H, D), 1) // DH
    row = lax.broadcasted_iota(jnp.int32, (H, D), 0)
    ind = (col == row).astype(jnp.float32)
    qsq = q * q
    qn2_ref[...] = lax.dot_general(ind, qsq, ((((1,), (1,)), ((), ()))),
                                   preferred_element_type=jnp.float32,
                                   precision=lax.Precision.HIGHEST)


def _qkv_call(x2d, wqkv, bqkv):
    grid = (T // TBLK,)
    return pl.pallas_call(
        _qkv_body,
        grid=grid,
        in_specs=[
            pl.BlockSpec((TBLK, D), lambda i: (i, 0)),
            pl.BlockSpec((D, 3 * D), lambda i: (0, 0)),
            pl.BlockSpec((1, 3 * D), lambda i: (0, 0)),
        ],
        out_specs=[
            pl.BlockSpec((H, TBLK, DH), lambda i: (0, i, 0)),
            pl.BlockSpec((TBLK, 2 * D), lambda i: (i, 0)),
            pl.BlockSpec((H, TBLK), lambda i: (0, i)),
        ],
        out_shape=[
            jax.ShapeDtypeStruct((H, T, DH), jnp.float32),
            jax.ShapeDtypeStruct((T, 2 * D), jnp.float32),
            jax.ShapeDtypeStruct((H, T), jnp.float32),
        ],
    )(x2d, wqkv, bqkv)


def _thresh_body(qn2_ref, thr_ref):
    bits = lax.bitcast_convert_type(qn2_ref[...], jnp.int32)

    def count_ge(b):
        return jnp.sum((bits >= b).astype(jnp.int32), axis=1, keepdims=True)

    lo = jnp.zeros((H, 1), jnp.int32)
    hi = jnp.full((H, 1), 0x7F800000, jnp.int32)

    def step(_, carry):
        lo, hi = carry
        mid = lo + (hi - lo) // 2
        ge = count_ge(mid) >= KEEP
        return jnp.where(ge, mid, lo), jnp.where(ge, hi, mid)

    lo, hi = lax.fori_loop(0, 31, step, (lo, hi))
    tau = lax.bitcast_convert_type(lo, jnp.float32)
    n_gt = jnp.sum((bits > lo).astype(jnp.int32), axis=1, keepdims=True)
    need = (KEEP - n_gt).astype(jnp.float32)
    cidx = lax.broadcasted_iota(jnp.int32, (H, 128), 1)
    thr_ref[...] = jnp.where(cidx == 0, tau, jnp.where(cidx == 1, need, 0.0))


def _thresh_call(qn2):
    return pl.pallas_call(
        _thresh_body,
        out_shape=jax.ShapeDtypeStruct((H, 128), jnp.float32),
    )(qn2)


def _sel_gather_body(qn2_hbm, thr_hbm, kvt_hbm, kvsel_hbm,
                     qn2_v, thr_v, idxf_v, eqf_v, idx_v, rows_v, sem):
    h = lax.axis_index("s") * 2 + lax.axis_index("c")

    @pl.when(h < H)
    def _():
        pltpu.sync_copy(qn2_hbm.at[h], qn2_v)
        pltpu.sync_copy(thr_hbm.at[h], thr_v)
        tvec = thr_v[pl.ds(0, 16)]
        tau_s = tvec[0]
        need = tvec.astype(jnp.int32)[1]
        zeros16 = jnp.zeros((16,), jnp.int32)
        iota16 = lax.iota(jnp.int32, 16)
        h_v = jnp.full((16,), h, jnp.int32)
        for c in range(IDXF // 16):
            idxf_v[pl.ds(c * 16, 16)] = zeros16
        for c in range(KP // 16):
            eqf_v[pl.ds(c * 16, 16)] = zeros16

        def cbody(c, carry):
            wr, eqw = carry
            vals = qn2_v[pl.ds(c * 16, 16)]
            neq = jnp.int32(0)
            for i in range(16):
                vi = vals[i]
                gi = (c * 16 + i) * H + h
                cgt = vi > tau_s
                off = lax.select_n(cgt, jnp.int32(KP), wr)
                idxf_v[pl.ds(off, 16)] = jnp.full((16,), gi, jnp.int32)
                wr = lax.select_n(cgt, wr, wr + 1)
                neq = lax.select_n(vi == tau_s, neq, neq + 1)

            @pl.when(neq > 0)
            def _eq():
                loc = eqw
                for i in range(16):
                    vi = vals[i]
                    gi = (c * 16 + i) * H + h
                    ceq = vi == tau_s
                    off = lax.select_n(ceq, jnp.int32(EQTRASH), loc)
                    eqf_v[pl.ds(off, 16)] = jnp.full((16,), gi, jnp.int32)
                    loc = lax.select_n(ceq, loc, loc + 1)

            return wr, eqw + neq

        wr, _ = lax.fori_loop(0, T // 16, cbody,
                              (jnp.int32(0), jnp.int32(0)))

        nchunks = lax.shift_right_logical(need + 15, 4)

        def apbody(c2, _):
            idxf_v[pl.ds(wr + c2 * 16, 16)] = eqf_v[pl.ds(c2 * 16, 16)]
            return 0

        lax.fori_loop(0, nchunks, apbody, 0)

        for j in range(KP // 128):
            for c in range(8):
                idx_v[j, pl.ds(c * 16, 16)] = idxf_v[pl.ds(j * 128 + c * 16, 16)]

        for j in range(KP // 128):
            pltpu.async_copy(kvt_hbm.at[idx_v.at[j]], rows_v.at[j % 2],
                             sem).wait()
            pltpu.sync_copy(rows_v.at[j % 2],
                            kvsel_hbm.at[h, pl.ds(j * 128, 128)])


def _sel_gather_call(qn2, thr, kvt):
    mesh = plsc.VectorSubcoreMesh(core_axis_name="c", subcore_axis_name="s",
                                  num_cores=2, num_subcores=16)
    fn = pl.kernel(
        _sel_gather_body,
        out_type=jax.ShapeDtypeStruct((H, KP, 2 * DH), jnp.float32),
        mesh=mesh,
        scratch_types=[
            pltpu.VMEM((T,), jnp.float32),
            pltpu.VMEM((128,), jnp.float32),
            pltpu.VMEM((IDXF,), jnp.int32),
            pltpu.VMEM((EQTRASH + 16,), jnp.int32),
            pltpu.VMEM((KP // 128, 128), jnp.int32),
            pltpu.VMEM((2, 128, 2 * DH), jnp.float32),
            pltpu.SemaphoreType.DMA,
        ],
    )
    return fn(qn2, thr, kvt)


def _attn_body(q_ref, kv_ref, w_ref, b_ref, out_ref):
    acc = jnp.broadcast_to(b_ref[...], (ABLK, D))
    colv = lax.broadcasted_iota(jnp.int32, (ABLK, KP), 1)
    for h in range(H):
        q = q_ref[h].astype(jnp.bfloat16)
        k = kv_ref[h][:, :DH].astype(jnp.bfloat16)
        v = kv_ref[h][:, DH:].astype(jnp.bfloat16)
        s = lax.dot_general(q, k, ((((1,), (1,)), ((), ()))),
                            preferred_element_type=jnp.float32)
        s = s * (1.0 / (DH ** 0.5))
        s = jnp.where(colv < KEEP, s, -1e30)
        m = jnp.max(s, axis=1, keepdims=True)
        e = jnp.exp(s - m)
        p = (e / jnp.sum(e, axis=1, keepdims=True)).astype(jnp.bfloat16)
        o = lax.dot_general(p, v, ((((1,), (0,)), ((), ()))),
                            preferred_element_type=jnp.float32)
        acc = acc + _dot(o, w_ref[pl.ds(h * DH, DH)], ((1,), (0,)))
    out_ref[...] = acc


def _attn_call(q3, kvsel, wproj, bproj):
    grid = (T // ABLK,)
    return pl.pallas_call(
        _attn_body,
        grid=grid,
        in_specs=[
            pl.BlockSpec((H, ABLK, DH), lambda i: (0, i, 0)),
            pl.BlockSpec((H, KP, 2 * DH), lambda i: (0, 0, 0)),
            pl.BlockSpec((D, D), lambda i: (0, 0)),
            pl.BlockSpec((1, D), lambda i: (0, 0)),
        ],
        out_specs=pl.BlockSpec((ABLK, D), lambda i: (i, 0)),
        out_shape=jax.ShapeDtypeStruct((T, D), jnp.float32),
    )(q3, kvsel, wproj, bproj)


def kernel(x, Wqkv, bqkv, Wproj, bproj):
    x2d = x.reshape(T, D)
    q3, kv2d, qn2 = _qkv_call(x2d, Wqkv, bqkv.reshape(1, 3 * D))
    thr = _thresh_call(qn2)
    kvt = kv2d.reshape(T * H, 2 * DH)
    kvsel = _sel_gather_call(qn2, thr, kvt)
    out = _attn_call(q3, kvsel, Wproj, bproj.reshape(1, D))
    return out.reshape(1, T, D)


# speedup vs baseline: 1.2000x; 1.2000x over previous
import functools

import jax
import jax.numpy as jnp
from jax import lax
from jax.experimental import pallas as pl
from jax.experimental.pallas import tpu as pltpu
from jax.experimental.pallas import tpu_sc as plsc

T, D, H = 8192, 768, 12
DH = D // H
KEEP = max(1, int(T * 0.1))
KP = 896
TBLK = 1024
ABLK = 1024
NW = 32
IDXF = 912
EQTRASH = T + 16

_PREC = lax.Precision.DEFAULT
_DOTF32 = jnp.float32


def _dot(a, b, dims):
    return lax.dot_general(a, b, (dims, ((), ())),
                           preferred_element_type=_DOTF32, precision=_PREC)


def _qkv_body(x_ref, w_ref, b_ref, q_ref, kv_ref, qn2_ref):
    x = x_ref[...]
    q = _dot(x, w_ref[:, :D], (((1,), (0,)))) + b_ref[:, :D]
    for h in range(H):
        q_ref[h] = q[:, h * DH:(h + 1) * DH]
    kv = lax.dot_general(x.astype(jnp.bfloat16),
                         w_ref[:, D:].astype(jnp.bfloat16),
                         ((((1,), (0,)), ((), ()))),
                         preferred_element_type=jnp.float32) + b_ref[:, D:]
    parts = []
    for h in range(H):
        parts.append(kv[:, h * DH:(h + 1) * DH])
        parts.append(kv[:, D + h * DH:D + (h + 1) * DH])
    kvp = jnp.concatenate(parts, axis=1)
    kv_ref[...] = kvp.reshape(TBLK * H, 2 * DH)
    col = lax.broadcasted_iota(jnp.int32, (H, D), 1) // DH
    row = lax.broadcasted_iota(jnp.int32, (H, D), 0)
    ind = (col == row).astype(jnp.float32)
    qsq = q * q
    qn2_ref[...] = lax.dot_general(ind, qsq, ((((1,), (1,)), ((), ()))),
                                   preferred_element_type=jnp.float32,
                                   precision=lax.Precision.HIGHEST)


def _qkv_call(x2d, wqkv, bqkv):
    grid = (T // TBLK,)
    return pl.pallas_call(
        _qkv_body,
        grid=grid,
        in_specs=[
            pl.BlockSpec((TBLK, D), lambda i: (i, 0)),
            pl.BlockSpec((D, 3 * D), lambda i: (0, 0)),
            pl.BlockSpec((1, 3 * D), lambda i: (0, 0)),
        ],
        out_specs=[
            pl.BlockSpec((H, TBLK, DH), lambda i: (0, i, 0)),
            pl.BlockSpec((H * TBLK, 2 * DH), lambda i: (i, 0)),
            pl.BlockSpec((H, TBLK), lambda i: (0, i)),
        ],
        out_shape=[
            jax.ShapeDtypeStruct((H, T, DH), jnp.float32),
            jax.ShapeDtypeStruct((T * H, 2 * DH), jnp.float32),
            jax.ShapeDtypeStruct((H, T), jnp.float32),
        ],
    )(x2d, wqkv, bqkv)


def _thresh_body(qn2_ref, thr_ref):
    bits = lax.bitcast_convert_type(qn2_ref[...], jnp.int32)

    def count_ge(b):
        return jnp.sum((bits >= b).astype(jnp.int32), axis=1, keepdims=True)

    lo = jnp.zeros((H, 1), jnp.int32)
    hi = jnp.full((H, 1), 0x7F800000, jnp.int32)

    def step(_, carry):
        lo, hi = carry
        mid = lo + (hi - lo) // 2
        ge = count_ge(mid) >= KEEP
        return jnp.where(ge, mid, lo), jnp.where(ge, hi, mid)

    lo, hi = lax.fori_loop(0, 31, step, (lo, hi))
    tau = lax.bitcast_convert_type(lo, jnp.float32)
    n_gt = jnp.sum((bits > lo).astype(jnp.int32), axis=1, keepdims=True)
    need = (KEEP - n_gt).astype(jnp.float32)
    cidx = lax.broadcasted_iota(jnp.int32, (H, 128), 1)
    thr_ref[...] = jnp.where(cidx == 0, tau, jnp.where(cidx == 1, need, 0.0))


def _thresh_call(qn2):
    return pl.pallas_call(
        _thresh_body,
        out_shape=jax.ShapeDtypeStruct((H, 128), jnp.float32),
    )(qn2)


def _sel_gather_body(qn2_hbm, thr_hbm, kvt_hbm, kvsel_hbm,
                     qn2_v, thr_v, idxf_v, eqf_v, idx_v, rows_v, sem):
    h = lax.axis_index("s") * 2 + lax.axis_index("c")

    @pl.when(h < H)
    def _():
        pltpu.sync_copy(qn2_hbm.at[h], qn2_v)
        pltpu.sync_copy(thr_hbm.at[h], thr_v)
        tvec = thr_v[pl.ds(0, 16)]
        tau_s = tvec[0]
        need = tvec.astype(jnp.int32)[1]
        zeros16 = jnp.zeros((16,), jnp.int32)
        iota16 = lax.iota(jnp.int32, 16)
        h_v = jnp.full((16,), h, jnp.int32)
        for c in range(IDXF // 16):
            idxf_v[pl.ds(c * 16, 16)] = zeros16
        for c in range(KP // 16):
            eqf_v[pl.ds(c * 16, 16)] = zeros16

        def cbody(c, carry):
            wr, eqw = carry
            vals = qn2_v[pl.ds(c * 16, 16)]
            neq = jnp.int32(0)
            for i in range(16):
                vi = vals[i]
                gi = (c * 16 + i) * H + h
                cgt = vi > tau_s
                off = lax.select_n(cgt, jnp.int32(KP), wr)
                idxf_v[pl.ds(off, 16)] = jnp.full((16,), gi, jnp.int32)
                wr = lax.select_n(cgt, wr, wr + 1)
                neq = lax.select_n(vi == tau_s, neq, neq + 1)

            @pl.when(neq > 0)
            def _eq():
                loc = eqw
                for i in range(16):
                    vi = vals[i]
                    gi = (c * 16 + i) * H + h
                    ceq = vi == tau_s
                    off = lax.select_n(ceq, jnp.int32(EQTRASH), loc)
                    eqf_v[pl.ds(off, 16)] = jnp.full((16,), gi, jnp.int32)
                    loc = lax.select_n(ceq, loc, loc + 1)

            return wr, eqw + neq

        wr, _ = lax.fori_loop(0, T // 16, cbody,
                              (jnp.int32(0), jnp.int32(0)))

        nchunks = lax.shift_right_logical(need + 15, 4)

        def apbody(c2, _):
            idxf_v[pl.ds(wr + c2 * 16, 16)] = eqf_v[pl.ds(c2 * 16, 16)]
            return 0

        lax.fori_loop(0, nchunks, apbody, 0)

        for j in range(KP // 128):
            for c in range(8):
                idx_v[j, pl.ds(c * 16, 16)] = idxf_v[pl.ds(j * 128 + c * 16, 16)]

        for j in range(KP // 128):
            pltpu.async_copy(kvt_hbm.at[idx_v.at[j]], rows_v.at[j % 2],
                             sem).wait()
            pltpu.sync_copy(rows_v.at[j % 2],
                            kvsel_hbm.at[h, pl.ds(j * 128, 128)])


def _sel_gather_call(qn2, thr, kvt):
    mesh = plsc.VectorSubcoreMesh(core_axis_name="c", subcore_axis_name="s",
                                  num_cores=2, num_subcores=16)
    fn = pl.kernel(
        _sel_gather_body,
        out_type=jax.ShapeDtypeStruct((H, KP, 2 * DH), jnp.float32),
        mesh=mesh,
        scratch_types=[
            pltpu.VMEM((T,), jnp.float32),
            pltpu.VMEM((128,), jnp.float32),
            pltpu.VMEM((IDXF,), jnp.int32),
            pltpu.VMEM((EQTRASH + 16,), jnp.int32),
            pltpu.VMEM((KP // 128, 128), jnp.int32),
            pltpu.VMEM((2, 128, 2 * DH), jnp.float32),
            pltpu.SemaphoreType.DMA,
        ],
    )
    return fn(qn2, thr, kvt)


def _attn_body(q_ref, kv_ref, w_ref, b_ref, out_ref):
    acc = jnp.broadcast_to(b_ref[...], (ABLK, D))
    colv = lax.broadcasted_iota(jnp.int32, (1, KP), 1)
    bias = jnp.where(colv < KEEP, 0.0, -1e30)
    for h in range(H):
        q = (q_ref[h] * (1.0 / (DH ** 0.5))).astype(jnp.bfloat16)
        k = kv_ref[h][:, :DH].astype(jnp.bfloat16)
        v = kv_ref[h][:, DH:].astype(jnp.bfloat16)
        s = lax.dot_general(q, k, ((((1,), (1,)), ((), ()))),
                            preferred_element_type=jnp.float32)
        s = s + bias
        m = jnp.max(s, axis=1, keepdims=True)
        e = jnp.exp(s - m)
        inv = 1.0 / jnp.sum(e, axis=1, keepdims=True)
        o = lax.dot_general(e.astype(jnp.bfloat16), v,
                            ((((1,), (0,)), ((), ()))),
                            preferred_element_type=jnp.float32) * inv
        acc = acc + _dot(o, w_ref[pl.ds(h * DH, DH)], ((1,), (0,)))
    out_ref[...] = acc


def _attn_call(q3, kvsel, wproj, bproj):
    grid = (T // ABLK,)
    return pl.pallas_call(
        _attn_body,
        grid=grid,
        in_specs=[
            pl.BlockSpec((H, ABLK, DH), lambda i: (0, i, 0)),
            pl.BlockSpec((H, KP, 2 * DH), lambda i: (0, 0, 0)),
            pl.BlockSpec((D, D), lambda i: (0, 0)),
            pl.BlockSpec((1, D), lambda i: (0, 0)),
        ],
        out_specs=pl.BlockSpec((ABLK, D), lambda i: (i, 0)),
        out_shape=jax.ShapeDtypeStruct((T, D), jnp.float32),
    )(q3, kvsel, wproj, bproj)


def kernel(x, Wqkv, bqkv, Wproj, bproj):
    x2d = x.reshape(T, D)
    q3, kvt, qn2 = _qkv_call(x2d, Wqkv, bqkv.reshape(1, 3 * D))
    thr = _thresh_call(qn2)
    kvsel = _sel_gather_call(qn2, thr, kvt)
    out = _attn_call(q3, kvsel, Wproj, bproj.reshape(1, D))
    return out.reshape(1, T, D)


# speedup vs baseline: 1.2310x; 1.0258x over previous
import functools

import jax
import jax.numpy as jnp
from jax import lax
from jax.experimental import pallas as pl
from jax.experimental.pallas import tpu as pltpu
from jax.experimental.pallas import tpu_sc as plsc

T, D, H = 8192, 768, 12
DH = D // H
KEEP = max(1, int(T * 0.1))
KP = 896
TBLK = 1024
ABLK = 1024
NW = 32
TH = T // 2
CNTOFF = KP
IDXTRASH = KP + 16
IDXF = KP + 32
EQTRASH = TH + 16

_PREC = lax.Precision.DEFAULT
_DOTF32 = jnp.float32


def _dot(a, b, dims):
    return lax.dot_general(a, b, (dims, ((), ())),
                           preferred_element_type=_DOTF32, precision=_PREC)


def _qkv_body(x_ref, w_ref, b_ref, q_ref, kv_ref, qn2_ref):
    x = x_ref[...]
    q = _dot(x, w_ref[:, :D], (((1,), (0,)))) + b_ref[:, :D]
    for h in range(H):
        q_ref[h] = q[:, h * DH:(h + 1) * DH]
    kv = lax.dot_general(x.astype(jnp.bfloat16),
                         w_ref[:, D:].astype(jnp.bfloat16),
                         ((((1,), (0,)), ((), ()))),
                         preferred_element_type=jnp.float32) + b_ref[:, D:]
    parts = []
    for h in range(H):
        parts.append(kv[:, h * DH:(h + 1) * DH])
        parts.append(kv[:, D + h * DH:D + (h + 1) * DH])
    kvp = jnp.concatenate(parts, axis=1)
    kv_ref[...] = kvp.reshape(TBLK * H, 2 * DH)
    col = lax.broadcasted_iota(jnp.int32, (H, D), 1) // DH
    row = lax.broadcasted_iota(jnp.int32, (H, D), 0)
    ind = (col == row).astype(jnp.float32)
    qsq = q * q
    qn2_ref[...] = lax.dot_general(ind, qsq, ((((1,), (1,)), ((), ()))),
                                   preferred_element_type=jnp.float32,
                                   precision=lax.Precision.HIGHEST)


def _qkv_call(x2d, wqkv, bqkv):
    grid = (T // TBLK,)
    return pl.pallas_call(
        _qkv_body,
        grid=grid,
        in_specs=[
            pl.BlockSpec((TBLK, D), lambda i: (i, 0)),
            pl.BlockSpec((D, 3 * D), lambda i: (0, 0)),
            pl.BlockSpec((1, 3 * D), lambda i: (0, 0)),
        ],
        out_specs=[
            pl.BlockSpec((H, TBLK, DH), lambda i: (0, i, 0)),
            pl.BlockSpec((H * TBLK, 2 * DH), lambda i: (i, 0)),
            pl.BlockSpec((H, TBLK), lambda i: (0, i)),
        ],
        out_shape=[
            jax.ShapeDtypeStruct((H, T, DH), jnp.float32),
            jax.ShapeDtypeStruct((T * H, 2 * DH), jnp.float32),
            jax.ShapeDtypeStruct((H, T), jnp.float32),
        ],
    )(x2d, wqkv, bqkv)


def _thresh_body(qn2_ref, thr_ref):
    bits = lax.bitcast_convert_type(qn2_ref[...], jnp.int32)

    def count_ge(b):
        return jnp.sum((bits >= b).astype(jnp.int32), axis=1, keepdims=True)

    lo = jnp.zeros((H, 1), jnp.int32)
    hi = jnp.full((H, 1), 0x7F800000, jnp.int32)

    def step(_, carry):
        lo, hi = carry
        mid = lo + (hi - lo) // 2
        ge = count_ge(mid) >= KEEP
        return jnp.where(ge, mid, lo), jnp.where(ge, hi, mid)

    lo, hi = lax.fori_loop(0, 31, step, (lo, hi))
    tau = lax.bitcast_convert_type(lo, jnp.float32)
    n_gt = jnp.sum((bits > lo).astype(jnp.int32), axis=1, keepdims=True)
    need = (KEEP - n_gt).astype(jnp.float32)
    h0 = bits[:, :TH]
    cg0 = jnp.sum((h0 > lo).astype(jnp.int32), axis=1,
                  keepdims=True).astype(jnp.float32)
    ce0 = jnp.sum((h0 == lo).astype(jnp.int32), axis=1,
                  keepdims=True).astype(jnp.float32)
    cgt = n_gt.astype(jnp.float32)
    cidx = lax.broadcasted_iota(jnp.int32, (H, 128), 1)
    out = jnp.where(cidx == 0, tau, 0.0)
    out = jnp.where(cidx == 1, need, out)
    out = jnp.where(cidx == 2, cg0, out)
    out = jnp.where(cidx == 3, ce0, out)
    out = jnp.where(cidx == 4, cgt, out)
    thr_ref[...] = out


def _thresh_call(qn2):
    return pl.pallas_call(
        _thresh_body,
        out_shape=jax.ShapeDtypeStruct((H, 128), jnp.float32),
    )(qn2)


def _sel_gather_body(qn2_hbm, thr_hbm, kvt_hbm, kvsel_hbm,
                     qn2_v, thr_v, idxf_v, eqf_v, idx_v, rows_v, shared_v,
                     pbuf_v, sem):
    sid = lax.axis_index("s")
    w = lax.axis_index("c") * 16 + sid
    h = lax.shift_right_logical(w, 1)
    half = lax.bitwise_and(w, 1)

    @pl.when(h < H)
    def _():
        pltpu.sync_copy(qn2_hbm.at[h, pl.ds(half * TH, TH)], qn2_v)
        pltpu.sync_copy(thr_hbm.at[h], thr_v)
        tvec = thr_v[pl.ds(0, 16)]
        tau_s = tvec[0]
        tvi = tvec.astype(jnp.int32)
        need = tvi[1]
        ce0 = tvi[3]
        is1 = half > 0
        k0 = jnp.minimum(need, ce0)
        my_take = lax.select_n(is1, k0, need - k0)
        tbase = half * TH
        zeros16 = jnp.zeros((16,), jnp.int32)
        for c in range(IDXF // 16):
            idxf_v[pl.ds(c * 16, 16)] = zeros16
        for c in range(KP // 16):
            eqf_v[pl.ds(c * 16, 16)] = zeros16

        def cbody(c, carry):
            wr, eqw = carry
            vals = qn2_v[pl.ds(c * 16, 16)]
            neq = jnp.int32(0)
            for i in range(16):
                vi = vals[i]
                gi = (tbase + c * 16 + i) * H + h
                cgtl = vi > tau_s
                off = lax.select_n(cgtl, jnp.int32(IDXTRASH), wr)
                idxf_v[pl.ds(off, 16)] = jnp.full((16,), gi, jnp.int32)
                wr = lax.select_n(cgtl, wr, wr + 1)
                neq = lax.select_n(vi == tau_s, neq, neq + 1)

            @pl.when(neq > 0)
            def _eq():
                loc = eqw
                for i in range(16):
                    vi = vals[i]
                    gi = (tbase + c * 16 + i) * H + h
                    ceq = vi == tau_s
                    off = lax.select_n(ceq, jnp.int32(EQTRASH), loc)
                    eqf_v[pl.ds(off, 16)] = jnp.full((16,), gi, jnp.int32)
                    loc = lax.select_n(ceq, loc, loc + 1)

            return wr, eqw + neq

        wr, _ = lax.fori_loop(0, TH // 16, cbody,
                              (jnp.int32(0), jnp.int32(0)))

        nchunks = lax.shift_right_logical(my_take + 15, 4)

        def apbody(c2, _):
            idxf_v[pl.ds(wr + c2 * 16, 16)] = eqf_v[pl.ds(c2 * 16, 16)]
            return 0

        lax.fori_loop(0, nchunks, apbody, 0)
        n_loc = wr + my_take
        idxf_v[pl.ds(CNTOFF, 16)] = jnp.full((16,), n_loc, jnp.int32)
        pltpu.sync_copy(idxf_v, shared_v.at[sid])

    plsc.subcore_barrier()

    @pl.when(jnp.logical_and(h < H, half == 0))
    def _gather():
        pltpu.sync_copy(shared_v.at[sid + 1], pbuf_v)
        myn = idxf_v[pl.ds(CNTOFF, 16)][0]
        pn = pbuf_v[pl.ds(CNTOFF, 16)][0]
        nchunks2 = lax.shift_right_logical(pn + 15, 4)

        def mbody(c2, _):
            idxf_v[pl.ds(myn + c2 * 16, 16)] = pbuf_v[pl.ds(c2 * 16, 16)]
            return 0

        lax.fori_loop(0, nchunks2, mbody, 0)

        for j in range(KP // 128):
            for c in range(8):
                idx_v[j, pl.ds(c * 16, 16)] = idxf_v[pl.ds(j * 128 + c * 16, 16)]

        for j in range(KP // 128):
            pltpu.async_copy(kvt_hbm.at[idx_v.at[j]], rows_v.at[j % 2],
                             sem).wait()
            pltpu.sync_copy(rows_v.at[j % 2],
                            kvsel_hbm.at[h, pl.ds(j * 128, 128)])


def _sel_gather_call(qn2, thr, kvt):
    mesh = plsc.VectorSubcoreMesh(core_axis_name="c", subcore_axis_name="s",
                                  num_cores=2, num_subcores=16)
    fn = pl.kernel(
        _sel_gather_body,
        out_type=jax.ShapeDtypeStruct((H, KP, 2 * DH), jnp.float32),
        mesh=mesh,
        scratch_types=[
            pltpu.VMEM((TH,), jnp.float32),
            pltpu.VMEM((128,), jnp.float32),
            pltpu.VMEM((IDXF,), jnp.int32),
            pltpu.VMEM((EQTRASH + 16,), jnp.int32),
            pltpu.VMEM((KP // 128, 128), jnp.int32),
            pltpu.VMEM((2, 128, 2 * DH), jnp.float32),
            pltpu.VMEM_SHARED((16, IDXF), jnp.int32),
            pltpu.VMEM((IDXF,), jnp.int32),
            pltpu.SemaphoreType.DMA,
        ],
    )
    return fn(qn2, thr, kvt)


def _attn_body(q_ref, kv_ref, w_ref, b_ref, out_ref):
    acc = jnp.broadcast_to(b_ref[...], (ABLK, D))
    colv = lax.broadcasted_iota(jnp.int32, (1, KP), 1)
    bias = jnp.where(colv < KEEP, 0.0, -1e30)
    rowm = lax.broadcasted_iota(jnp.int32, (KP, 1), 0) < KEEP
    for h in range(H):
        q = (q_ref[h] * (1.0 / (DH ** 0.5))).astype(jnp.bfloat16)
        k = jnp.where(rowm, kv_ref[h][:, :DH], 0.0).astype(jnp.bfloat16)
        v = jnp.where(rowm, kv_ref[h][:, DH:], 0.0).astype(jnp.bfloat16)
        s = lax.dot_general(q, k, ((((1,), (1,)), ((), ()))),
                            preferred_element_type=jnp.float32)
        s = s + bias
        m = jnp.max(s, axis=1, keepdims=True)
        e = jnp.exp(s - m)
        inv = 1.0 / jnp.sum(e, axis=1, keepdims=True)
        o = lax.dot_general(e.astype(jnp.bfloat16), v,
                            ((((1,), (0,)), ((), ()))),
                            preferred_element_type=jnp.float32) * inv
        acc = acc + _dot(o, w_ref[pl.ds(h * DH, DH)], ((1,), (0,)))
    out_ref[...] = acc


def _attn_call(q3, kvsel, wproj, bproj):
    grid = (T // ABLK,)
    return pl.pallas_call(
        _attn_body,
        grid=grid,
        in_specs=[
            pl.BlockSpec((H, ABLK, DH), lambda i: (0, i, 0)),
            pl.BlockSpec((H, KP, 2 * DH), lambda i: (0, 0, 0)),
            pl.BlockSpec((D, D), lambda i: (0, 0)),
            pl.BlockSpec((1, D), lambda i: (0, 0)),
        ],
        out_specs=pl.BlockSpec((ABLK, D), lambda i: (i, 0)),
        out_shape=jax.ShapeDtypeStruct((T, D), jnp.float32),
    )(q3, kvsel, wproj, bproj)


def kernel(x, Wqkv, bqkv, Wproj, bproj):
    x2d = x.reshape(T, D)
    q3, kvt, qn2 = _qkv_call(x2d, Wqkv, bqkv.reshape(1, 3 * D))
    thr = _thresh_call(qn2)
    kvsel = _sel_gather_call(qn2, thr, kvt)
    out = _attn_call(q3, kvsel, Wproj, bproj.reshape(1, D))
    return out.reshape(1, T, D)
